# Initial kernel scaffold; baseline (speedup 1.0000x reference)
import jax
import jax.numpy as jnp
from jax.experimental import pallas as pl


def kernel(features, emb_table):
    raise NotImplementedError("write your pallas kernel here")


# speedup vs baseline: 1.1563x; 1.1563x over previous
import functools

import jax
import jax.numpy as jnp
from jax import lax
from jax.experimental import pallas as pl
from jax.experimental.pallas import tpu as pltpu
from jax.experimental.pallas import tpu_sc as plsc

VOCAB = 100000
D = 64
B = 4096
HIST = 50
HIST_PAD = 56
NC, NS = 2, 16
NW = NC * NS
ITEMS_PER_W = B // NW
CHUNK_ITEMS = 2
IDX_PER_CHUNK = CHUNK_ITEMS * HIST_PAD
NCHUNKS = ITEMS_PER_W // CHUNK_ITEMS
LANES = 16
DJ = D // LANES


@functools.partial(
    pl.kernel,
    mesh=plsc.VectorSubcoreMesh(core_axis_name="c", subcore_axis_name="s"),
    out_type=jax.ShapeDtypeStruct((B, HIST, D), jnp.float32),
    scratch_types=[
        pltpu.VMEM((ITEMS_PER_W * HIST_PAD,), jnp.int32),
        pltpu.VMEM((IDX_PER_CHUNK, D), jnp.float32),
        pltpu.SemaphoreType.DMA,
    ],
    compiler_params=pltpu.CompilerParams(use_tc_tiling_on_sc=False),
)
def _sc_kernel(feat_hbm, table_hbm, out_hbm, idx_v, rows_v, gsem):
    cid = lax.axis_index("c")
    sid = lax.axis_index("s")
    wid = sid * NC + cid
    item0 = wid * ITEMS_PER_W
    pltpu.sync_copy(
        feat_hbm.at[pl.ds(item0 * HIST_PAD, ITEMS_PER_W * HIST_PAD)], idx_v)

    def chunk_body(c, carry):
        pltpu.async_copy(
            table_hbm.at[idx_v.at[pl.ds(c * IDX_PER_CHUNK, IDX_PER_CHUNK)]],
            rows_v, gsem).wait()

        for it in range(CHUNK_ITEMS):
            r0 = it * HIST_PAD
            zero = jnp.zeros((LANES,), jnp.float32)

            def p1(l, acc, r0=r0):
                return tuple(
                    acc[j] + jnp.abs(rows_v[r0 + l, pl.ds(j * LANES, LANES)])
                    for j in range(DJ))

            acc = lax.fori_loop(0, HIST, p1, (zero,) * DJ)
            rn = tuple(1.0 / jnp.maximum(acc[j], 1e-12) for j in range(DJ))

            def p2(l, cc, r0=r0, rn=rn):
                for j in range(DJ):
                    x = rows_v[r0 + l, pl.ds(j * LANES, LANES)]
                    y = 1.0 / (1.0 + jnp.exp(-(x * rn[j])))
                    rows_v[r0 + l, pl.ds(j * LANES, LANES)] = y
                return cc

            lax.fori_loop(0, HIST, p2, 0)
            b = item0 + c * CHUNK_ITEMS + it
            pltpu.sync_copy(rows_v.at[pl.ds(r0, HIST)], out_hbm.at[b])
        return carry

    lax.fori_loop(0, NCHUNKS, chunk_body, 0)


def kernel(features, emb_table):
    feats = features.astype(jnp.int32)
    feats_p = jnp.pad(feats, ((0, 0), (0, HIST_PAD - HIST))).reshape(-1)
    return _sc_kernel(feats_p, emb_table)
